# baseline (speedup 1.0000x reference)
import functools

import jax
import jax.numpy as jnp
from jax import lax
from jax.experimental import pallas as pl
from jax.experimental.pallas import tpu as pltpu
from jax.experimental.pallas import tpu_sc as plsc

_N = 10000
_E = 320000
_D = 128

_NC = 2
_NS = 16
_NW = _NC * _NS
_GROUP = 128
_GPT = 80
_NGP = _NW * _GPT
_EPAD = _NGP * _GROUP
_NPAIR = _GPT // 2
_RPT = 632
_NPAD = _NS * _RPT


def _sc_spmm(x, packed, w2, zeros):
    mesh = plsc.VectorSubcoreMesh(core_axis_name="c", subcore_axis_name="s")

    @functools.partial(
        pl.kernel,
        mesh=mesh,
        out_type=jax.ShapeDtypeStruct((_NC, _NPAD, _D), jnp.float32),
        scratch_types=[
            pltpu.VMEM((2, 2, _GROUP), jnp.int32),
            pltpu.VMEM((2, 2, _GROUP), jnp.int32),
            pltpu.VMEM((2, _GROUP), jnp.float32),
            pltpu.VMEM((2, _GROUP), jnp.float32),
            pltpu.VMEM((_GROUP, _D), jnp.float32),
            pltpu.VMEM((_GROUP, _D), jnp.float32),
            pltpu.VMEM_SHARED((_NPAD, _D), jnp.float32),
            pltpu.SemaphoreType.DMA,
            pltpu.SemaphoreType.DMA,
            pltpu.SemaphoreType.DMA,
            pltpu.SemaphoreType.DMA,
        ],
    )
    def k(x_hbm, pk_hbm, w_hbm, z_hbm, out_hbm,
          pka, pkb, wa, wb, rows0, rows1, acc_sh, spka, spkb, sg0, sg1):
        cid = lax.axis_index("c")
        sid = lax.axis_index("s")
        wid = sid * _NC + cid
        pbase = wid * _NPAIR

        def pk_fetch(pair, buf, wbuf, sem):
            pltpu.async_copy(
                pk_hbm.at[pl.ds((pbase + pair) * 2, 2)], buf, sem)
            pltpu.async_copy(
                w_hbm.at[pl.ds((pbase + pair) * 2, 2)], wbuf, sem)

        def pk_wait(buf, wbuf, sem):
            pltpu.make_async_copy(pk_hbm.at[pl.ds(0, 2)], buf, sem).wait()
            pltpu.make_async_copy(w_hbm.at[pl.ds(0, 2)], wbuf, sem).wait()

        def gather(buf_pk, i, rows, sem):
            pltpu.async_copy(x_hbm.at[buf_pk.at[i, 0]], rows, sem)

        def wait(sem, buf):
            pltpu.make_async_copy(x_hbm.at[pka.at[0, 0]], buf, sem).wait()

        def scale(rows, wbuf, i):
            def escale(s, c2):
                wv16 = wbuf[i, pl.ds(s * 16, 16)]
                for j in range(16):
                    e = s * 16 + j
                    wv = jnp.full((16,), wv16[j], dtype=jnp.float32)
                    for dd in range(_D // 16):
                        sl = pl.ds(dd * 16, 16)
                        rows[e, sl] = rows[e, sl] * wv
                return c2

            lax.fori_loop(0, _GROUP // 16, escale, 0)

        def scatter(rows, buf_pk, i):
            pltpu.sync_copy(rows, acc_sh.at[buf_pk.at[i, 1]], add=True)

        pk_fetch(0, pka, wa, spka)
        pk_wait(pka, wa, spka)
        pk_fetch(1, pkb, wb, spkb)
        gather(pka, 0, rows0, sg0)
        gather(pka, 1, rows1, sg1)
        pltpu.sync_copy(z_hbm, acc_sh.at[pl.ds(sid * _RPT, _RPT)])
        plsc.subcore_barrier()

        def body4(u, carry):
            wait(sg0, rows0)
            scale(rows0, wa, 0)
            scatter(rows0, pka, 0)
            pk_wait(pkb, wb, spkb)
            gather(pkb, 0, rows0, sg0)
            wait(sg1, rows1)
            scale(rows1, wa, 1)
            scatter(rows1, pka, 1)
            pk_fetch(2 * u + 2, pka, wa, spka)
            gather(pkb, 1, rows1, sg1)
            wait(sg0, rows0)
            scale(rows0, wb, 0)
            scatter(rows0, pkb, 0)
            pk_wait(pka, wa, spka)
            gather(pka, 0, rows0, sg0)
            wait(sg1, rows1)
            scale(rows1, wb, 1)
            scatter(rows1, pkb, 1)
            pk_fetch(2 * u + 3, pkb, wb, spkb)
            gather(pka, 1, rows1, sg1)
            return carry

        lax.fori_loop(0, _NPAIR // 2 - 1, body4, 0)

        wait(sg0, rows0)
        scale(rows0, wa, 0)
        scatter(rows0, pka, 0)
        pk_wait(pkb, wb, spkb)
        gather(pkb, 0, rows0, sg0)
        wait(sg1, rows1)
        scale(rows1, wa, 1)
        scatter(rows1, pka, 1)
        gather(pkb, 1, rows1, sg1)
        wait(sg0, rows0)
        scale(rows0, wb, 0)
        scatter(rows0, pkb, 0)
        wait(sg1, rows1)
        scale(rows1, wb, 1)
        scatter(rows1, pkb, 1)

        plsc.subcore_barrier()
        pltpu.sync_copy(acc_sh.at[pl.ds(sid * _RPT, _RPT)],
                        out_hbm.at[cid, pl.ds(sid * _RPT, _RPT)])

    return k(x, packed, w2, zeros)


def _add_body(a_ref, o_ref):
    o_ref[...] = a_ref[0] + a_ref[1]


def _combine(partials):
    blk = 1000
    return pl.pallas_call(
        _add_body,
        grid=(_N // blk,),
        in_specs=[pl.BlockSpec((_NC, blk, _D), lambda i: (0, i, 0))],
        out_specs=pl.BlockSpec((blk, _D), lambda i: (i, 0)),
        out_shape=jax.ShapeDtypeStruct((_N, _D), jnp.float32),
    )(partials)


def kernel(x, edge_index, edge_weight):
    pad = _EPAD - _E
    col = jnp.concatenate([edge_index[1], jnp.zeros((pad,), jnp.int32)])
    row = jnp.concatenate([edge_index[0], jnp.zeros((pad,), jnp.int32)])
    packed = jnp.stack([
        col.reshape(_NGP, _GROUP),
        row.reshape(_NGP, _GROUP)], axis=1)
    w2 = jnp.concatenate(
        [edge_weight, jnp.zeros((pad,), jnp.float32)]).reshape(_NGP, _GROUP)
    zeros = jnp.zeros((_RPT, _D), jnp.float32)
    partials = _sc_spmm(x, packed, w2, zeros)
    return _combine(partials[:, :_N])


# speedup vs baseline: 1.1674x; 1.1674x over previous
import functools

import jax
import jax.numpy as jnp
from jax import lax
from jax.experimental import pallas as pl
from jax.experimental.pallas import tpu as pltpu
from jax.experimental.pallas import tpu_sc as plsc

_N = 10000
_E = 320000
_D = 128

_NC = 2
_NS = 16
_NW = _NC * _NS
_GROUP = 128
_GPT = 80
_NGP = _NW * _GPT
_EPAD = _NGP * _GROUP
_NPAIR = _GPT // 2
_RPT = 632
_NPAD = _NS * _RPT


def _sc_spmm(x, packed, w2, zeros):
    mesh = plsc.VectorSubcoreMesh(core_axis_name="c", subcore_axis_name="s")

    @functools.partial(
        pl.kernel,
        mesh=mesh,
        out_type=jax.ShapeDtypeStruct((_NC, _NPAD, _D), jnp.float32),
        scratch_types=[
            pltpu.VMEM((2, 2, _GROUP), jnp.int32),
            pltpu.VMEM((2, 2, _GROUP), jnp.int32),
            pltpu.VMEM((2, _GROUP), jnp.float32),
            pltpu.VMEM((2, _GROUP), jnp.float32),
            pltpu.VMEM((_GROUP, _D), jnp.float32),
            pltpu.VMEM((_GROUP, _D), jnp.float32),
            pltpu.VMEM_SHARED((_NPAD, _D), jnp.float32),
            pltpu.SemaphoreType.DMA,
            pltpu.SemaphoreType.DMA,
            pltpu.SemaphoreType.DMA,
            pltpu.SemaphoreType.DMA,
        ],
    )
    def k(x_hbm, pk_hbm, w_hbm, z_hbm, out_hbm,
          pka, pkb, wa, wb, rows0, rows1, acc_sh, spka, spkb, sg0, sg1):
        cid = lax.axis_index("c")
        sid = lax.axis_index("s")
        wid = sid * _NC + cid
        def pk_fetch(pair, buf, wbuf, sem):
            gp = wid + pair * _NW
            pltpu.async_copy(pk_hbm.at[pl.ds(gp * 2, 2)], buf, sem)
            pltpu.async_copy(w_hbm.at[gp], wbuf, sem)

        def pk_wait(buf, wbuf, sem):
            pltpu.make_async_copy(pk_hbm.at[pl.ds(0, 2)], buf, sem).wait()
            pltpu.make_async_copy(w_hbm.at[0], wbuf, sem).wait()

        def gather(buf_pk, i, rows, sem):
            pltpu.async_copy(x_hbm.at[buf_pk.at[i, 0]], rows, sem)

        def wait(sem, buf):
            pltpu.make_async_copy(x_hbm.at[pka.at[0, 0]], buf, sem).wait()

        def scale(rows, wbuf, i):
            def escale(s, c2):
                wv16 = wbuf[i, pl.ds(s * 16, 16)]
                for j in range(16):
                    e = s * 16 + j
                    wv = jnp.full((16,), wv16[j], dtype=jnp.float32)
                    for dd in range(_D // 16):
                        sl = pl.ds(dd * 16, 16)
                        rows[e, sl] = rows[e, sl] * wv
                return c2

            lax.fori_loop(0, _GROUP // 16, escale, 0)

        def scatter(rows, buf_pk, i):
            pltpu.sync_copy(rows, acc_sh.at[buf_pk.at[i, 1]], add=True)

        pk_fetch(0, pka, wa, spka)
        pk_wait(pka, wa, spka)
        pk_fetch(1, pkb, wb, spkb)
        gather(pka, 0, rows0, sg0)
        gather(pka, 1, rows1, sg1)
        pltpu.sync_copy(z_hbm, acc_sh.at[pl.ds(sid * _RPT, _RPT)])
        plsc.subcore_barrier()

        def body4(u, carry):
            wait(sg0, rows0)
            scale(rows0, wa, 0)
            scatter(rows0, pka, 0)
            pk_wait(pkb, wb, spkb)
            gather(pkb, 0, rows0, sg0)
            wait(sg1, rows1)
            scale(rows1, wa, 1)
            scatter(rows1, pka, 1)
            pk_fetch(2 * u + 2, pka, wa, spka)
            gather(pkb, 1, rows1, sg1)
            wait(sg0, rows0)
            scale(rows0, wb, 0)
            scatter(rows0, pkb, 0)
            pk_wait(pka, wa, spka)
            gather(pka, 0, rows0, sg0)
            wait(sg1, rows1)
            scale(rows1, wb, 1)
            scatter(rows1, pkb, 1)
            pk_fetch(2 * u + 3, pkb, wb, spkb)
            gather(pka, 1, rows1, sg1)
            return carry

        lax.fori_loop(0, _NPAIR // 2 - 1, body4, 0)

        wait(sg0, rows0)
        scale(rows0, wa, 0)
        scatter(rows0, pka, 0)
        pk_wait(pkb, wb, spkb)
        gather(pkb, 0, rows0, sg0)
        wait(sg1, rows1)
        scale(rows1, wa, 1)
        scatter(rows1, pka, 1)
        gather(pkb, 1, rows1, sg1)
        wait(sg0, rows0)
        scale(rows0, wb, 0)
        scatter(rows0, pkb, 0)
        wait(sg1, rows1)
        scale(rows1, wb, 1)
        scatter(rows1, pkb, 1)

        plsc.subcore_barrier()
        pltpu.sync_copy(acc_sh.at[pl.ds(sid * _RPT, _RPT)],
                        out_hbm.at[cid, pl.ds(sid * _RPT, _RPT)])

    return k(x, packed, w2, zeros)


def _add_body(a_ref, o_ref):
    o_ref[...] = a_ref[0] + a_ref[1]


def _combine(partials):
    blk = 1000
    return pl.pallas_call(
        _add_body,
        grid=(_N // blk,),
        in_specs=[pl.BlockSpec((_NC, blk, _D), lambda i: (0, i, 0))],
        out_specs=pl.BlockSpec((blk, _D), lambda i: (i, 0)),
        out_shape=jax.ShapeDtypeStruct((_N, _D), jnp.float32),
    )(partials)


def kernel(x, edge_index, edge_weight):
    pad = _EPAD - _E
    col = jnp.concatenate([edge_index[1], jnp.zeros((pad,), jnp.int32)])
    row = jnp.concatenate([edge_index[0], jnp.zeros((pad,), jnp.int32)])
    packed = jnp.stack([
        col.reshape(_NGP, _GROUP),
        row.reshape(_NGP, _GROUP)], axis=1)
    w2 = jnp.concatenate(
        [edge_weight, jnp.zeros((pad,), jnp.float32)]).reshape(
            _NGP // 2, 2, _GROUP)
    zeros = jnp.zeros((_RPT, _D), jnp.float32)
    partials = _sc_spmm(x, packed, w2, zeros)
    return _combine(partials[:, :_N])
